# Initial kernel scaffold; baseline (speedup 1.0000x reference)
import jax
import jax.numpy as jnp
from jax.experimental import pallas as pl


def kernel(p0_box, p0_ctr, p0_cls, p1_box, p1_ctr, p1_cls, p2_box, p2_ctr, p2_cls, a0, a1, a2, image_size):
    raise NotImplementedError("write your pallas kernel here")


# speedup vs baseline: 18.7005x; 18.7005x over previous
import jax
import jax.numpy as jnp
from jax.experimental import pallas as pl
from jax.experimental.pallas import tpu as pltpu

_CONF = 0.2
_IOU = 0.6
_MAXDET = 100
_BS = 8
_NS = (4096, 1024, 256)
_NTOT = 5376


def _decode_level(b_ref, c_ref, k_ref, a_ref, pln, off, n):
    ax1 = a_ref[0:1, :]
    ay1 = a_ref[1:2, :]
    ax2 = a_ref[2:3, :]
    ay2 = a_ref[3:4, :]
    px = 0.5 * (ax1 + ax2)
    py = 0.5 * (ay1 + ay2)
    pw = ax2 - ax1
    ph = ay2 - ay1

    tx = b_ref[:, 0, :]
    ty = b_ref[:, 1, :]
    tr = b_ref[:, 2, :]
    tb = b_ref[:, 3, :]
    x1 = px - tx * pw
    y1 = py - ty * ph
    x2 = px + tr * pw
    y2 = py + tb * ph
    cx = 0.5 * (x1 + x2)
    cy = 0.5 * (y1 + y2)
    w = x2 - x1
    h = y2 - y1
    hw = 0.5 * w
    hh = 0.5 * h
    bx1 = cx - hw
    by1 = cy - hh
    bx2 = cx + hw
    by2 = cy + hh

    cl = k_ref[:]
    mx = jnp.max(cl, axis=1)
    iotac = jax.lax.broadcasted_iota(jnp.int32, cl.shape, 1)
    cid = jnp.min(jnp.where(cl == mx[:, None, :], iotac, 80), axis=1)

    obj = jax.nn.sigmoid(c_ref[:])
    mcs = jax.nn.sigmoid(mx)
    conf = jnp.sqrt(obj * mcs)
    sc = jnp.where(conf > _CONF, conf, jnp.float32(-1.0))

    clf = cid.astype(jnp.float32)
    o = clf * 4096.0
    ox1 = bx1 + o
    oy1 = by1 + o
    ox2 = bx2 + o
    oy2 = by2 + o
    areas = (ox2 - ox1) * (oy2 - oy1)

    sl = slice(off, off + n)
    pln[0, :, sl] = sc
    pln[1, :, sl] = ox1
    pln[2, :, sl] = oy1
    pln[3, :, sl] = ox2
    pln[4, :, sl] = oy2
    pln[5, :, sl] = areas
    pln[6, :, sl] = bx1
    pln[7, :, sl] = by1
    pln[8, :, sl] = bx2
    pln[9, :, sl] = by2
    pln[10, :, sl] = clf


def _fused_body(b0, c0, k0, b1, c1, k1, b2, c2, k2, a0, a1, a2, out_ref, pln):
    _decode_level(b0, c0, k0, a0, pln, 0, _NS[0])
    _decode_level(b1, c1, k1, a1, pln, _NS[0], _NS[1])
    _decode_level(b2, c2, k2, a2, pln, _NS[0] + _NS[1], _NS[2])

    iota = jax.lax.broadcasted_iota(jnp.int32, (_BS, _NTOT), 1)

    def body(t, carry):
        sc = pln[0, :, :]
        m = jnp.max(sc, axis=1, keepdims=True)
        eq = sc == m
        idx = jnp.min(jnp.where(eq, iota, _NTOT), axis=1, keepdims=True)
        oh = iota == idx
        g = jnp.sum(jnp.where(oh[None], pln[1:11, :, :], 0.0), axis=2,
                    keepdims=True)
        bb0 = g[0]
        bb1 = g[1]
        bb2 = g[2]
        bb3 = g[3]
        ba = g[4]
        valid = m > 0.0

        ix1 = jnp.maximum(bb0, pln[1, :, :])
        iy1 = jnp.maximum(bb1, pln[2, :, :])
        ix2 = jnp.minimum(bb2, pln[3, :, :])
        iy2 = jnp.minimum(bb3, pln[4, :, :])
        inter = jnp.maximum(ix2 - ix1, 0.0) * jnp.maximum(iy2 - iy1, 0.0)
        iou = inter / (ba + pln[5, :, :] - inter + 1e-9)
        sup = (iou >= _IOU) & valid
        nsc = jnp.where(sup, -1.0, sc)
        nsc = jnp.where(oh, -1.0, nsc)
        pln[0, :, :] = nsc

        zero = jnp.zeros_like(m)
        det = jnp.concatenate(
            [jnp.where(valid, g[5], zero),
             jnp.where(valid, g[6], zero),
             jnp.where(valid, g[7], zero),
             jnp.where(valid, g[8], zero),
             jnp.where(valid, m, zero),
             jnp.where(valid, g[9], -1.0)], axis=1)
        out_ref[pl.ds(t, 1)] = det[None]
        return 0

    jax.lax.fori_loop(0, _MAXDET, body, 0)


@jax.jit
def _run(p0_box, p0_ctr, p0_cls, p1_box, p1_ctr, p1_cls,
         p2_box, p2_ctr, p2_cls, a0, a1, a2):
    n0, n1, n2 = _NS
    args = (
        p0_box.reshape(_BS, 4, n0), p0_ctr.reshape(_BS, n0),
        p0_cls.reshape(_BS, 80, n0),
        p1_box.reshape(_BS, 4, n1), p1_ctr.reshape(_BS, n1),
        p1_cls.reshape(_BS, 80, n1),
        p2_box.reshape(_BS, 4, n2), p2_ctr.reshape(_BS, n2),
        p2_cls.reshape(_BS, 80, n2),
        a0.T, a1.T, a2.T,
    )
    out = pl.pallas_call(
        _fused_body,
        out_shape=jax.ShapeDtypeStruct((_MAXDET, _BS, 6), jnp.float32),
        scratch_shapes=[pltpu.VMEM((11, _BS, _NTOT), jnp.float32)],
    )(*args)
    return jnp.transpose(out, (1, 0, 2))


def kernel(p0_box, p0_ctr, p0_cls, p1_box, p1_ctr, p1_cls,
           p2_box, p2_ctr, p2_cls, a0, a1, a2, image_size):
    return _run(p0_box, p0_ctr, p0_cls, p1_box, p1_ctr, p1_cls,
                p2_box, p2_ctr, p2_cls, a0, a1, a2)


# speedup vs baseline: 19.1905x; 1.0262x over previous
import jax
import jax.numpy as jnp
from jax.experimental import pallas as pl
from jax.experimental.pallas import tpu as pltpu

_CONF = 0.2
_IOU = 0.6
_MAXDET = 100
_BS = 8
_NS = (4096, 1024, 256)
_NTOT = 5376


def _decode_level(b_ref, c_ref, k_ref, a_ref, pln, off, n):
    ax1 = a_ref[0:1, :]
    ay1 = a_ref[1:2, :]
    ax2 = a_ref[2:3, :]
    ay2 = a_ref[3:4, :]
    px = 0.5 * (ax1 + ax2)
    py = 0.5 * (ay1 + ay2)
    pw = ax2 - ax1
    ph = ay2 - ay1

    tx = b_ref[:, 0, :]
    ty = b_ref[:, 1, :]
    tr = b_ref[:, 2, :]
    tb = b_ref[:, 3, :]
    x1 = px - tx * pw
    y1 = py - ty * ph
    x2 = px + tr * pw
    y2 = py + tb * ph
    cx = 0.5 * (x1 + x2)
    cy = 0.5 * (y1 + y2)
    w = x2 - x1
    h = y2 - y1
    hw = 0.5 * w
    hh = 0.5 * h
    bx1 = cx - hw
    by1 = cy - hh
    bx2 = cx + hw
    by2 = cy + hh

    cl = k_ref[:]
    mx = jnp.max(cl, axis=1)
    iotac = jax.lax.broadcasted_iota(jnp.int32, cl.shape, 1)
    cid = jnp.min(jnp.where(cl == mx[:, None, :], iotac, 80), axis=1)

    obj = jax.nn.sigmoid(c_ref[:])
    mcs = jax.nn.sigmoid(mx)
    conf = jnp.sqrt(obj * mcs)
    sc = jnp.where(conf > _CONF, conf, jnp.float32(-1.0))

    clf = cid.astype(jnp.float32)
    o = clf * 4096.0
    ox1 = bx1 + o
    oy1 = by1 + o
    ox2 = bx2 + o
    oy2 = by2 + o
    areas = (ox2 - ox1) * (oy2 - oy1)

    sl = slice(off, off + n)
    pln[0, :, sl] = sc
    pln[1, :, sl] = ox1
    pln[2, :, sl] = oy1
    pln[3, :, sl] = ox2
    pln[4, :, sl] = oy2
    pln[5, :, sl] = areas
    pln[6, :, sl] = bx1
    pln[7, :, sl] = by1
    pln[8, :, sl] = bx2
    pln[9, :, sl] = by2
    pln[10, :, sl] = clf


def _fused_body(b0, c0, k0, b1, c1, k1, b2, c2, k2, a0, a1, a2, out_ref, pln):
    _decode_level(b0, c0, k0, a0, pln, 0, _NS[0])
    _decode_level(b1, c1, k1, a1, pln, _NS[0], _NS[1])
    _decode_level(b2, c2, k2, a2, pln, _NS[0] + _NS[1], _NS[2])

    iota = jax.lax.broadcasted_iota(jnp.int32, (_BS, _NTOT), 1)
    m0 = jnp.max(pln[0, :, :], axis=1, keepdims=True)

    def body(t, m):
        sc = pln[0, :, :]
        eq = sc == m
        idx = jnp.min(jnp.where(eq, iota, _NTOT), axis=1, keepdims=True)
        oh = iota == idx
        g = jnp.sum(jnp.where(oh[None], pln[6:11, :, :], 0.0), axis=2,
                    keepdims=True)
        woff = g[4] * 4096.0
        bb0 = g[0] + woff
        bb1 = g[1] + woff
        bb2 = g[2] + woff
        bb3 = g[3] + woff
        ba = (bb2 - bb0) * (bb3 - bb1)
        valid = m > 0.0

        ix1 = jnp.maximum(bb0, pln[1, :, :])
        iy1 = jnp.maximum(bb1, pln[2, :, :])
        ix2 = jnp.minimum(bb2, pln[3, :, :])
        iy2 = jnp.minimum(bb3, pln[4, :, :])
        inter = jnp.maximum(ix2 - ix1, 0.0) * jnp.maximum(iy2 - iy1, 0.0)
        iou = inter / (ba + pln[5, :, :] - inter + 1e-9)
        sup = ((iou >= _IOU) & valid) | oh
        nsc = jnp.where(sup, -1.0, sc)
        pln[0, :, :] = nsc
        mnew = jnp.max(nsc, axis=1, keepdims=True)

        zero = jnp.zeros_like(m)
        det = jnp.concatenate(
            [jnp.where(valid, g[0], zero),
             jnp.where(valid, g[1], zero),
             jnp.where(valid, g[2], zero),
             jnp.where(valid, g[3], zero),
             jnp.where(valid, m, zero),
             jnp.where(valid, g[4], -1.0)], axis=1)
        out_ref[pl.ds(t, 1)] = det[None]
        return mnew

    jax.lax.fori_loop(0, _MAXDET, body, m0)


@jax.jit
def _run(p0_box, p0_ctr, p0_cls, p1_box, p1_ctr, p1_cls,
         p2_box, p2_ctr, p2_cls, a0, a1, a2):
    n0, n1, n2 = _NS
    args = (
        p0_box.reshape(_BS, 4, n0), p0_ctr.reshape(_BS, n0),
        p0_cls.reshape(_BS, 80, n0),
        p1_box.reshape(_BS, 4, n1), p1_ctr.reshape(_BS, n1),
        p1_cls.reshape(_BS, 80, n1),
        p2_box.reshape(_BS, 4, n2), p2_ctr.reshape(_BS, n2),
        p2_cls.reshape(_BS, 80, n2),
        a0.T, a1.T, a2.T,
    )
    out = pl.pallas_call(
        _fused_body,
        out_shape=jax.ShapeDtypeStruct((_MAXDET, _BS, 6), jnp.float32),
        scratch_shapes=[pltpu.VMEM((11, _BS, _NTOT), jnp.float32)],
    )(*args)
    return jnp.transpose(out, (1, 0, 2))


def kernel(p0_box, p0_ctr, p0_cls, p1_box, p1_ctr, p1_cls,
           p2_box, p2_ctr, p2_cls, a0, a1, a2, image_size):
    return _run(p0_box, p0_ctr, p0_cls, p1_box, p1_ctr, p1_cls,
                p2_box, p2_ctr, p2_cls, a0, a1, a2)
